# baseline (speedup 1.0000x reference)
import functools

import jax
import jax.numpy as jnp
from jax import lax
from jax.experimental import pallas as pl
from jax.experimental.pallas import tpu as pltpu
from jax.experimental.pallas import tpu_sc as plsc

B = 4096
F = 26
V = 100000
D = 32
TOT = F * D

NC = 2
NS = 16
NW = NC * NS
ROWS = B * F
RPW = ROWS // NW
CHUNK = 128
ECHUNK = CHUNK * D
NCH = RPW // CHUNK
NBUF = 3


def _gather_body(tab_ref, eidx_ref, out_ref, idx_v, rows_v, sem_i, sem_g):
    wid = lax.axis_index("s") * NC + lax.axis_index("c")
    ebase = wid * RPW * D

    def idx_copy(j, buf):
        return pltpu.make_async_copy(
            eidx_ref.at[pl.ds(ebase + j * ECHUNK, ECHUNK)],
            idx_v.at[buf], sem_i)

    def gather_copy(j, buf):
        return pltpu.make_async_copy(
            tab_ref.at[idx_v.at[buf]],
            rows_v.at[pl.ds(j * ECHUNK, ECHUNK)], sem_g)

    idx_copy(0, 0).start()
    idx_copy(1, 1).start()

    def chunk_step(j, _):
        buf = j % NBUF
        idx_copy(j, buf).wait()
        gather_copy(j, buf).start()

        @pl.when(j >= 1)
        def _():
            gather_copy(j - 1, (j - 1) % NBUF).wait()

        @pl.when(j + 2 < NCH)
        def _():
            idx_copy(j + 2, (j + 2) % NBUF).start()

        return 0

    lax.fori_loop(0, NCH, chunk_step, 0)
    gather_copy(NCH - 1, (NCH - 1) % NBUF).wait()
    pltpu.sync_copy(rows_v, out_ref.at[pl.ds(ebase, RPW * D)])


@functools.cache
def _make_gather():
    return pl.kernel(
        _gather_body,
        out_type=jax.ShapeDtypeStruct((ROWS * D,), jnp.float32),
        mesh=plsc.VectorSubcoreMesh(core_axis_name="c", subcore_axis_name="s",
                                    num_cores=NC, num_subcores=NS),
        scratch_types=[
            pltpu.VMEM((NBUF, ECHUNK), jnp.int32),
            pltpu.VMEM((RPW * D,), jnp.float32),
            pltpu.SemaphoreType.DMA,
            pltpu.SemaphoreType.DMA,
        ],
        compiler_params=pltpu.CompilerParams(use_tc_tiling_on_sc=False),
    )


def _tower_body(x_ref, g_ref, bb_ref, w1_ref, b1_ref, w2_ref, b2_ref,
                w3_ref, b3_ref, out_ref):
    x = x_ref[...]
    mu = jnp.mean(x, axis=0, keepdims=True)
    xc = x - mu
    var = jnp.mean(xc * xc, axis=0, keepdims=True)
    xn = xc * (g_ref[...] * lax.rsqrt(var + 1e-5)) + bb_ref[...]
    h = jnp.dot(xn, w1_ref[...], preferred_element_type=jnp.float32)
    h = jnp.maximum(h + b1_ref[...], 0.0)
    h = jnp.dot(h, w2_ref[...], preferred_element_type=jnp.float32)
    h = jnp.maximum(h + b2_ref[...], 0.0)
    out = jnp.dot(h, w3_ref[...], preferred_element_type=jnp.float32)
    out_ref[...] = out + b3_ref[...]


def _tower(x, g, bb, w1, b1, w2, b2, w3, b3):
    return pl.pallas_call(
        _tower_body,
        out_shape=jax.ShapeDtypeStruct((B, 128), jnp.float32),
    )(x, g, bb, w1, b1, w2, b2, w3, b3)


def kernel(sparse, tables, bn_gamma, bn_beta, W1, b1, W2, b2, W3, b3):
    flat = sparse + jnp.arange(F, dtype=jnp.int32) * V
    eidx = (flat[..., None] * D + jnp.arange(D, dtype=jnp.int32)).reshape(-1)
    tab1 = tables.reshape(F * V * D)
    gathered = _make_gather()(tab1, eidx)
    x = gathered.reshape(B, TOT)
    return _tower(
        x,
        bn_gamma.reshape(1, TOT),
        bn_beta.reshape(1, TOT),
        W1, b1.reshape(1, 512),
        W2, b2.reshape(1, 256),
        W3, b3.reshape(1, 128),
    )


# speedup vs baseline: 1.1340x; 1.1340x over previous
import functools

import jax
import jax.numpy as jnp
from jax import lax
from jax.experimental import pallas as pl
from jax.experimental.pallas import tpu as pltpu
from jax.experimental.pallas import tpu_sc as plsc

B = 4096
F = 26
V = 100000
D = 32
TOT = F * D

NC = 2
NS = 16
NW = NC * NS
ROWS = B * F
CHUNK = 128
BCHUNKS = B // CHUNK
UNITS = F * BCHUNKS
UPW = UNITS // NW


def _gather_body(tab_ref, vidx_ref, oidx_ref, out_ref,
                 vidx_v, oidx_v, buf_v, sem_g, sem_s):
    wid = lax.axis_index("s") * NC + lax.axis_index("c")
    u0 = wid * UPW
    pltpu.sync_copy(vidx_ref.at[pl.ds(u0, UPW)], vidx_v)
    pltpu.sync_copy(oidx_ref.at[pl.ds(u0, UPW)], oidx_v)

    def gather(k, b):
        f = (u0 + k) // BCHUNKS
        return pltpu.make_async_copy(
            tab_ref.at[f].at[vidx_v.at[k]], buf_v.at[b], sem_g)

    def scatter(k, b):
        return pltpu.make_async_copy(
            buf_v.at[b], out_ref.at[oidx_v.at[k]], sem_s)

    gather(0, 0).start()
    for k in range(UPW):
        gather(k, k % 2).wait()
        if k + 1 < UPW:
            gather(k + 1, (k + 1) % 2).start()
        scatter(k, k % 2).start()
        if k >= 1:
            scatter(k - 1, (k - 1) % 2).wait()
    scatter(UPW - 1, (UPW - 1) % 2).wait()


@functools.cache
def _make_gather():
    return pl.kernel(
        _gather_body,
        out_type=jax.ShapeDtypeStruct((ROWS, D), jnp.float32),
        mesh=plsc.VectorSubcoreMesh(core_axis_name="c", subcore_axis_name="s",
                                    num_cores=NC, num_subcores=NS),
        scratch_types=[
            pltpu.VMEM((UPW, CHUNK), jnp.int32),
            pltpu.VMEM((UPW, CHUNK), jnp.int32),
            pltpu.VMEM((2, CHUNK, D), jnp.float32),
            pltpu.SemaphoreType.DMA,
            pltpu.SemaphoreType.DMA,
        ],
        compiler_params=pltpu.CompilerParams(use_tc_tiling_on_sc=False),
    )


def _tower_body(x_ref, g_ref, bb_ref, w1_ref, b1_ref, w2_ref, b2_ref,
                w3_ref, b3_ref, out_ref):
    x = x_ref[...]
    mu = jnp.mean(x, axis=0, keepdims=True)
    xc = x - mu
    var = jnp.mean(xc * xc, axis=0, keepdims=True)
    xn = xc * (g_ref[...] * lax.rsqrt(var + 1e-5)) + bb_ref[...]
    h = jnp.dot(xn, w1_ref[...], preferred_element_type=jnp.float32)
    h = jnp.maximum(h + b1_ref[...], 0.0)
    h = jnp.dot(h, w2_ref[...], preferred_element_type=jnp.float32)
    h = jnp.maximum(h + b2_ref[...], 0.0)
    out = jnp.dot(h, w3_ref[...], preferred_element_type=jnp.float32)
    out_ref[...] = out + b3_ref[...]


def _tower(x, g, bb, w1, b1, w2, b2, w3, b3):
    return pl.pallas_call(
        _tower_body,
        out_shape=jax.ShapeDtypeStruct((B, 128), jnp.float32),
    )(x, g, bb, w1, b1, w2, b2, w3, b3)


def kernel(sparse, tables, bn_gamma, bn_beta, W1, b1, W2, b2, W3, b3):
    vidx = sparse.T.reshape(UNITS, CHUNK)
    brow = jnp.arange(B, dtype=jnp.int32).reshape(BCHUNKS, CHUNK)
    oidx = (brow[None, :, :] * F
            + jnp.arange(F, dtype=jnp.int32)[:, None, None]
            ).reshape(UNITS, CHUNK)
    gathered = _make_gather()(tables, vidx, oidx)
    x = gathered.reshape(B, TOT)
    return _tower(
        x,
        bn_gamma.reshape(1, TOT),
        bn_beta.reshape(1, TOT),
        W1, b1.reshape(1, 512),
        W2, b2.reshape(1, 256),
        W3, b3.reshape(1, 128),
    )


# speedup vs baseline: 2.1259x; 1.8748x over previous
import functools

import jax
import jax.numpy as jnp
from jax import lax
from jax.experimental import pallas as pl
from jax.experimental.pallas import tpu as pltpu
from jax.experimental.pallas import tpu_sc as plsc

B = 4096
F = 26
V = 100000
D = 32
TOT = F * D

NC = 2
NS = 16
NW = NC * NS
CPW = TOT // NW


def _gather_body(tab_ref, sid_ref, out_ref, ids_v, col_v, sem_g):
    wid = lax.axis_index("s") * NC + lax.axis_index("c")
    u0 = wid * CPW
    f0 = u0 // D
    f1 = (u0 + CPW - 1) // D
    pltpu.sync_copy(sid_ref.at[f0], ids_v.at[0])
    pltpu.sync_copy(sid_ref.at[f1], ids_v.at[1])

    def gather(k, b):
        u = u0 + k
        f = u // D
        c = u % D
        lane = jnp.where(f == f0, 0, 1)
        return pltpu.make_async_copy(
            tab_ref.at[f].at[c].at[ids_v.at[lane]], col_v.at[b], sem_g)

    gather(0, 0).start()
    for k in range(CPW):
        gather(k, k % 2).wait()
        if k + 1 < CPW:
            gather(k + 1, (k + 1) % 2).start()
        pltpu.sync_copy(col_v.at[k % 2], out_ref.at[u0 + k])


@functools.cache
def _make_gather():
    return pl.kernel(
        _gather_body,
        out_type=jax.ShapeDtypeStruct((TOT, B), jnp.float32),
        mesh=plsc.VectorSubcoreMesh(core_axis_name="c", subcore_axis_name="s",
                                    num_cores=NC, num_subcores=NS),
        scratch_types=[
            pltpu.VMEM((2, B), jnp.int32),
            pltpu.VMEM((2, B), jnp.float32),
            pltpu.SemaphoreType.DMA,
        ],
        compiler_params=pltpu.CompilerParams(use_tc_tiling_on_sc=False),
    )


def _tower_body(xt_ref, g_ref, bb_ref, w1_ref, b1_ref, w2_ref, b2_ref,
                w3_ref, b3_ref, out_ref):
    xt = xt_ref[...]
    mu = jnp.mean(xt, axis=1, keepdims=True)
    xc = xt - mu
    var = jnp.mean(xc * xc, axis=1, keepdims=True)
    xn = xc * (g_ref[...] * lax.rsqrt(var + 1e-5)) + bb_ref[...]
    h = lax.dot_general(xn, w1_ref[...], (((0,), (0,)), ((), ())),
                        preferred_element_type=jnp.float32)
    h = jnp.maximum(h + b1_ref[...], 0.0)
    h = jnp.dot(h, w2_ref[...], preferred_element_type=jnp.float32)
    h = jnp.maximum(h + b2_ref[...], 0.0)
    out = jnp.dot(h, w3_ref[...], preferred_element_type=jnp.float32)
    out_ref[...] = out + b3_ref[...]


def _tower(xt, g, bb, w1, b1, w2, b2, w3, b3):
    return pl.pallas_call(
        _tower_body,
        out_shape=jax.ShapeDtypeStruct((B, 128), jnp.float32),
    )(xt, g, bb, w1, b1, w2, b2, w3, b3)


def kernel(sparse, tables, bn_gamma, bn_beta, W1, b1, W2, b2, W3, b3):
    tabt = jnp.transpose(tables, (0, 2, 1))
    sid = sparse.T
    xt = _make_gather()(tabt, sid)
    return _tower(
        xt,
        bn_gamma.reshape(TOT, 1),
        bn_beta.reshape(TOT, 1),
        W1, b1.reshape(1, 512),
        W2, b2.reshape(1, 256),
        W3, b3.reshape(1, 128),
    )
